# baseline (speedup 1.0000x reference)
import jax
import jax.numpy as jnp
from jax.experimental import pallas as pl
from jax.experimental.pallas import tpu as pltpu

_BM = 400
_NBUF = 3
_NOBUF = 2


def _body(a_hbm, x_hbm, w_ref, b_ref, o_hbm, abuf, xbuf, obuf, s_ref,
          in_sem, x_sem, out_sem):
    n = a_hbm.shape[0]
    nsteps = n // _BM

    def copy_in(k):
        return pltpu.make_async_copy(
            a_hbm.at[pl.ds(k * _BM, _BM), :], abuf.at[k % _NBUF],
            in_sem.at[k % _NBUF])

    def copy_out(k):
        return pltpu.make_async_copy(
            obuf.at[k % _NOBUF], o_hbm.at[pl.ds(k * _BM, _BM), :],
            out_sem.at[k % _NOBUF])

    for k in range(_NBUF):
        copy_in(k).start()
    xcopy = pltpu.make_async_copy(x_hbm, xbuf, x_sem)
    xcopy.start()
    xcopy.wait()
    x = xbuf[...].astype(jnp.bfloat16)
    w = w_ref[...].astype(jnp.bfloat16)
    s_ref[...] = jnp.dot(x, w, preferred_element_type=jnp.float32).astype(
        jnp.bfloat16)
    bias = b_ref[...]

    for k in range(nsteps):
        copy_in(k).wait()
        if k >= _NOBUF:
            copy_out(k - _NOBUF).wait()
        a = abuf[k % _NBUF].astype(jnp.bfloat16)
        obuf[k % _NOBUF] = (
            jnp.dot(a, s_ref[...], preferred_element_type=jnp.float32) + bias)
        copy_out(k).start()
        if k + _NBUF < nsteps:
            copy_in(k + _NBUF).start()

    for k in range(max(nsteps - _NOBUF, 0), nsteps):
        copy_out(k).wait()


def kernel(features, adjacency, weight, bias):
    n, d_in = features.shape
    d_out = weight.shape[1]
    bias2 = bias.reshape(1, d_out)

    out = pl.pallas_call(
        _body,
        in_specs=[
            pl.BlockSpec(memory_space=pl.ANY),
            pl.BlockSpec(memory_space=pl.ANY),
            pl.BlockSpec(memory_space=pltpu.VMEM),
            pl.BlockSpec(memory_space=pltpu.VMEM),
        ],
        out_specs=pl.BlockSpec(memory_space=pl.ANY),
        out_shape=jax.ShapeDtypeStruct((n, d_out), jnp.float32),
        scratch_shapes=[
            pltpu.VMEM((_NBUF, _BM, n), jnp.float32),
            pltpu.VMEM((n, d_in), jnp.float32),
            pltpu.VMEM((_NOBUF, _BM, d_out), jnp.float32),
            pltpu.VMEM((n, d_out), jnp.bfloat16),
            pltpu.SemaphoreType.DMA((_NBUF,)),
            pltpu.SemaphoreType.DMA,
            pltpu.SemaphoreType.DMA((_NOBUF,)),
        ],
    )(adjacency, features, weight, bias2)
    return out


# speedup vs baseline: 1.0220x; 1.0220x over previous
import jax
import jax.numpy as jnp
from jax.experimental import pallas as pl
from jax.experimental.pallas import tpu as pltpu

_BM = 200
_NBUF = 4
_NOBUF = 2


def _body(a_hbm, x_hbm, w_ref, b_ref, o_hbm, abuf, xbuf, obuf, s_ref,
          in_sem, x_sem, out_sem):
    n = a_hbm.shape[0]
    nsteps = n // _BM

    def copy_in(k):
        return pltpu.make_async_copy(
            a_hbm.at[pl.ds(k * _BM, _BM), :], abuf.at[k % _NBUF],
            in_sem.at[k % _NBUF])

    def copy_out(k):
        return pltpu.make_async_copy(
            obuf.at[k % _NOBUF], o_hbm.at[pl.ds(k * _BM, _BM), :],
            out_sem.at[k % _NOBUF])

    for k in range(_NBUF):
        copy_in(k).start()
    xcopy = pltpu.make_async_copy(x_hbm, xbuf, x_sem)
    xcopy.start()
    xcopy.wait()
    x = xbuf[...].astype(jnp.bfloat16)
    w = w_ref[...].astype(jnp.bfloat16)
    s_ref[...] = jnp.dot(x, w, preferred_element_type=jnp.float32).astype(
        jnp.bfloat16)
    bias = b_ref[...]

    for k in range(nsteps):
        copy_in(k).wait()
        if k >= _NOBUF:
            copy_out(k - _NOBUF).wait()
        a = abuf[k % _NBUF].astype(jnp.bfloat16)
        obuf[k % _NOBUF] = (
            jnp.dot(a, s_ref[...], preferred_element_type=jnp.float32) + bias)
        copy_out(k).start()
        if k + _NBUF < nsteps:
            copy_in(k + _NBUF).start()

    for k in range(max(nsteps - _NOBUF, 0), nsteps):
        copy_out(k).wait()


def kernel(features, adjacency, weight, bias):
    n, d_in = features.shape
    d_out = weight.shape[1]
    bias2 = bias.reshape(1, d_out)

    out = pl.pallas_call(
        _body,
        in_specs=[
            pl.BlockSpec(memory_space=pl.ANY),
            pl.BlockSpec(memory_space=pl.ANY),
            pl.BlockSpec(memory_space=pltpu.VMEM),
            pl.BlockSpec(memory_space=pltpu.VMEM),
        ],
        out_specs=pl.BlockSpec(memory_space=pl.ANY),
        out_shape=jax.ShapeDtypeStruct((n, d_out), jnp.float32),
        scratch_shapes=[
            pltpu.VMEM((_NBUF, _BM, n), jnp.float32),
            pltpu.VMEM((n, d_in), jnp.float32),
            pltpu.VMEM((_NOBUF, _BM, d_out), jnp.float32),
            pltpu.VMEM((n, d_out), jnp.bfloat16),
            pltpu.SemaphoreType.DMA((_NBUF,)),
            pltpu.SemaphoreType.DMA,
            pltpu.SemaphoreType.DMA((_NOBUF,)),
        ],
    )(adjacency, features, weight, bias2)
    return out


# speedup vs baseline: 1.0734x; 1.0503x over previous
import jax
import jax.numpy as jnp
from jax.experimental import pallas as pl
from jax.experimental.pallas import tpu as pltpu


def _fused_body(a_ref, x_ref, w_ref, b_ref, o_ref, s_ref):
    @pl.when(pl.program_id(0) == 0)
    def _():
        x = x_ref[...].astype(jnp.bfloat16)
        w = w_ref[...].astype(jnp.bfloat16)
        s_ref[...] = jnp.dot(x, w, preferred_element_type=jnp.float32).astype(
            jnp.bfloat16
        )

    a = a_ref[...].astype(jnp.bfloat16)
    acc = jnp.dot(a, s_ref[...], preferred_element_type=jnp.float32)
    o_ref[...] = acc + b_ref[...]


def kernel(features, adjacency, weight, bias):
    n, d_in = features.shape
    d_out = weight.shape[1]
    bias2 = bias.reshape(1, d_out)

    bm = 480
    out = pl.pallas_call(
        _fused_body,
        grid=(pl.cdiv(n, bm),),
        in_specs=[
            pl.BlockSpec((bm, n), lambda i: (i, 0)),
            pl.BlockSpec((n, d_in), lambda i: (0, 0)),
            pl.BlockSpec((d_in, d_out), lambda i: (0, 0)),
            pl.BlockSpec((1, d_out), lambda i: (0, 0)),
        ],
        out_specs=pl.BlockSpec((bm, d_out), lambda i: (i, 0)),
        out_shape=jax.ShapeDtypeStruct((n, d_out), jnp.float32),
        scratch_shapes=[pltpu.VMEM((n, d_out), jnp.bfloat16)],
        compiler_params=pltpu.CompilerParams(
            dimension_semantics=("arbitrary",),
        ),
    )(adjacency, features, weight, bias2)
    return out


# speedup vs baseline: 1.0803x; 1.0064x over previous
import jax
import jax.numpy as jnp
from jax.experimental import pallas as pl
from jax.experimental.pallas import tpu as pltpu


def _fused_body(a_ref, x_ref, w_ref, b_ref, o_ref, s_ref):
    @pl.when(pl.program_id(0) == 0)
    def _():
        x = x_ref[...].astype(jnp.bfloat16)
        w = w_ref[...].astype(jnp.bfloat16)
        s_ref[...] = jnp.dot(x, w, preferred_element_type=jnp.float32).astype(
            jnp.bfloat16
        )

    a = a_ref[...].astype(jnp.bfloat16)
    acc = jnp.dot(a, s_ref[...], preferred_element_type=jnp.float32)
    o_ref[...] = acc + b_ref[...]


def kernel(features, adjacency, weight, bias):
    n, d_in = features.shape
    d_out = weight.shape[1]
    bias2 = bias.reshape(1, d_out)

    bm = 320
    out = pl.pallas_call(
        _fused_body,
        grid=(pl.cdiv(n, bm),),
        in_specs=[
            pl.BlockSpec((bm, n), lambda i: (i, 0)),
            pl.BlockSpec((n, d_in), lambda i: (0, 0)),
            pl.BlockSpec((d_in, d_out), lambda i: (0, 0)),
            pl.BlockSpec((1, d_out), lambda i: (0, 0)),
        ],
        out_specs=pl.BlockSpec((bm, d_out), lambda i: (i, 0)),
        out_shape=jax.ShapeDtypeStruct((n, d_out), jnp.float32),
        scratch_shapes=[pltpu.VMEM((n, d_out), jnp.bfloat16)],
        compiler_params=pltpu.CompilerParams(
            dimension_semantics=("arbitrary",),
        ),
    )(adjacency, features, weight, bias2)
    return out
